# baseline (speedup 1.0000x reference)
import functools

import jax
import jax.numpy as jnp
from jax.experimental import pallas as pl
from jax.experimental.pallas import tpu as pltpu

_TILE_OUT = 1024


def _linear_kernel(x_ref, w_ref, b_ref, o_ref):
    acc = jax.lax.dot_general(
        x_ref[...],
        w_ref[...],
        dimension_numbers=(((1,), (1,)), ((), ())),
        preferred_element_type=jnp.float32,
    )
    o_ref[...] = acc + b_ref[...]


@jax.jit
def kernel(x, W, b):
    batch, in_features = x.shape
    out_features = W.shape[0]
    grid = pl.cdiv(out_features, _TILE_OUT)
    b2 = b.reshape(1, out_features)
    return pl.pallas_call(
        _linear_kernel,
        grid=(grid,),
        in_specs=[
            pl.BlockSpec((batch, in_features), lambda i: (0, 0)),
            pl.BlockSpec((_TILE_OUT, in_features), lambda i: (i, 0)),
            pl.BlockSpec((1, _TILE_OUT), lambda i: (0, i)),
        ],
        out_specs=pl.BlockSpec((batch, _TILE_OUT), lambda i: (0, i)),
        out_shape=jax.ShapeDtypeStruct((batch, out_features), jnp.float32),
        compiler_params=pltpu.CompilerParams(
            dimension_semantics=("arbitrary",),
        ),
    )(x, W, b2)


# speedup vs baseline: 1.1731x; 1.1731x over previous
import functools

import jax
import jax.numpy as jnp
from jax.experimental import pallas as pl
from jax.experimental.pallas import tpu as pltpu

_TILE_OUT = 2048


def _linear_kernel(x_ref, w_ref, b_ref, o_ref):
    acc = jax.lax.dot_general(
        x_ref[...],
        w_ref[...],
        dimension_numbers=(((1,), (1,)), ((), ())),
        preferred_element_type=jnp.float32,
    )
    o_ref[...] = acc + b_ref[...]


@jax.jit
def kernel(x, W, b):
    batch, in_features = x.shape
    out_features = W.shape[0]
    grid = pl.cdiv(out_features, _TILE_OUT)
    b2 = b.reshape(1, out_features)
    return pl.pallas_call(
        _linear_kernel,
        grid=(grid,),
        in_specs=[
            pl.BlockSpec((batch, in_features), lambda i: (0, 0)),
            pl.BlockSpec((_TILE_OUT, in_features), lambda i: (i, 0)),
            pl.BlockSpec((1, _TILE_OUT), lambda i: (0, i)),
        ],
        out_specs=pl.BlockSpec((batch, _TILE_OUT), lambda i: (0, i)),
        out_shape=jax.ShapeDtypeStruct((batch, out_features), jnp.float32),
        compiler_params=pltpu.CompilerParams(
            dimension_semantics=("parallel",),
        ),
    )(x, W, b2)


# speedup vs baseline: 1.1953x; 1.0189x over previous
import functools

import jax
import jax.numpy as jnp
from jax.experimental import pallas as pl
from jax.experimental.pallas import tpu as pltpu

_TILE_OUT = 2048


def _linear_kernel(x_ref, w_ref, b_ref, o_ref):
    acc = jax.lax.dot_general(
        x_ref[...],
        w_ref[...],
        dimension_numbers=(((1,), (1,)), ((), ())),
        preferred_element_type=jnp.float32,
    )
    o_ref[...] = acc + b_ref[...][None, :]


@jax.jit
def kernel(x, W, b):
    batch, in_features = x.shape
    out_features = W.shape[0]
    grid = pl.cdiv(out_features, _TILE_OUT)
    return pl.pallas_call(
        _linear_kernel,
        grid=(grid,),
        in_specs=[
            pl.BlockSpec((batch, in_features), lambda i: (0, 0)),
            pl.BlockSpec((_TILE_OUT, in_features), lambda i: (i, 0)),
            pl.BlockSpec((_TILE_OUT,), lambda i: (i,)),
        ],
        out_specs=pl.BlockSpec((batch, _TILE_OUT), lambda i: (0, i)),
        out_shape=jax.ShapeDtypeStruct((batch, out_features), jnp.float32),
        compiler_params=pltpu.CompilerParams(
            dimension_semantics=("parallel",),
        ),
    )(x, W, b)
